# baseline (speedup 1.0000x reference)
import functools

import jax
import jax.numpy as jnp
from jax import lax
from jax.experimental import pallas as pl
from jax.experimental.pallas import tpu as pltpu
from jax.experimental.pallas import tpu_sc as plsc

_REP = 128
_E = 320000
_NSEG = 20000
_R0 = 100000
_R1 = 120000

_BSEG = 80


def _mmf():
    return functools.partial(
        lax.dot_general,
        dimension_numbers=(((1,), (0,)), ((), ())),
        preferred_element_type=jnp.float32,
    )


def _family_body(size):
    brows = _BSEG * size

    def body(c_ref, g_ref, w1_ref, b1_ref, w2_ref, b2_ref,
             a1n_ref, a1s_ref, a1b_ref, a2n_ref, a2s_ref, a2b_ref, lm_ref):
        mm = _mmf()
        c = c_ref[...]
        g = g_ref[...]
        ri = lax.broadcasted_iota(jnp.int32, (brows, brows), 0) // size
        ci = lax.broadcasted_iota(jnp.int32, (brows, brows), 1) // size
        p = (ri == ci).astype(jnp.float32)
        w1 = w1_ref[...]
        hp = (mm(c, w1[0:128]) + mm(mm(p, c), w1[128:256])
              + mm(g, w1[256:384]) + mm(mm(p, g), w1[384:512]) + b1_ref[...])
        h = mm(jnp.maximum(hp, 0.0), w2_ref[...]) + b2_ref[...]
        a = jnp.maximum(
            mm(h, a1n_ref[...]) + mm(mm(p, h), a1s_ref[...]) + a1b_ref[...], 0.0)
        o = jnp.maximum(
            mm(a, a2n_ref[...]) + mm(mm(p, a), a2s_ref[...]) + a2b_ref[...], 0.0)
        lm_ref[:, 0:128] = o
        lm_ref[:, 128:256] = mm(p, o)

    return body


def _family_call(size, rows, c, g, w1, b1, w2, b2, a1n, a1s, a1b, a2n, a2s, a2b):
    brows = _BSEG * size
    row_spec = pl.BlockSpec((brows, _REP), lambda i: (i, 0))

    def wspec(shape):
        return pl.BlockSpec(shape, lambda i: (0,) * len(shape))

    return pl.pallas_call(
        _family_body(size),
        grid=(_NSEG // _BSEG,),
        in_specs=[row_spec, row_spec,
                  wspec((4 * _REP, 2 * _REP)), wspec((1, 2 * _REP)),
                  wspec((2 * _REP, _REP)), wspec((1, _REP)),
                  wspec((_REP, 2 * _REP)), wspec((_REP, 2 * _REP)),
                  wspec((1, 2 * _REP)),
                  wspec((2 * _REP, _REP)), wspec((2 * _REP, _REP)),
                  wspec((1, _REP))],
        out_specs=pl.BlockSpec((brows, 2 * _REP), lambda i: (i, 0)),
        out_shape=jax.ShapeDtypeStruct((rows, 2 * _REP), jnp.float32),
    )(c, g, w1, b1.reshape(1, -1), w2, b2.reshape(1, -1),
      a1n, a1s, a1b.reshape(1, -1), a2n, a2s, a2b.reshape(1, -1))


_BE = 512


def _edge_body(e_ref, m_ref, w1_ref, b1_ref, w2_ref, b2_ref, o_ref):
    mm = _mmf()
    w1 = w1_ref[...]
    hp = mm(e_ref[...], w1[0:128]) + mm(m_ref[...], w1[128:384]) + b1_ref[...]
    o_ref[...] = mm(jnp.maximum(hp, 0.0), w2_ref[...]) + b2_ref[...]


def _edge_call(edge_rep, c2e, w1, b1, w2, b2):
    return pl.pallas_call(
        _edge_body,
        grid=(_E // _BE,),
        in_specs=[pl.BlockSpec((_BE, _REP), lambda i: (i, 0)),
                  pl.BlockSpec((_BE, 2 * _REP), lambda i: (i, 0)),
                  pl.BlockSpec((3 * _REP, 2 * _REP), lambda i: (0, 0)),
                  pl.BlockSpec((1, 2 * _REP), lambda i: (0, 0)),
                  pl.BlockSpec((2 * _REP, _REP), lambda i: (0, 0)),
                  pl.BlockSpec((1, _REP), lambda i: (0, 0))],
        out_specs=pl.BlockSpec((_BE, _REP), lambda i: (i, 0)),
        out_shape=jax.ShapeDtypeStruct((_E, _REP), jnp.float32),
    )(edge_rep, c2e, w1, b1.reshape(1, -1), w2, b2.reshape(1, -1))


_GN = 225280
_GPT = _GN // 32
_GCH = 128
_GNCH = _GPT // _GCH


def _sc_gather(table, idx_all):
    mesh = plsc.VectorSubcoreMesh(core_axis_name="c", subcore_axis_name="s")

    @functools.partial(
        pl.kernel, mesh=mesh,
        compiler_params=pltpu.CompilerParams(needs_layout_passes=False),
        out_type=jax.ShapeDtypeStruct((_GN, _REP), jnp.float32),
        scratch_types=[pltpu.VMEM((_GPT,), jnp.int32),
                       pltpu.VMEM((_GCH,), jnp.int32),
                       pltpu.VMEM((_GCH, _REP), jnp.float32),
                       pltpu.SemaphoreType.DMA],
    )
    def gk(table_h, idx_h, out_h, idx_v, idxb, buf, sem):
        wid = lax.axis_index("s") * 2 + lax.axis_index("c")
        base = wid * _GPT
        pltpu.sync_copy(idx_h.at[pl.ds(base, _GPT)], idx_v)

        def step(i, carry):
            for k in range(_GCH // 16):
                idxb[pl.ds(k * 16, 16)] = idx_v[pl.ds(i * _GCH + k * 16, 16)]
            pltpu.async_copy(table_h.at[idxb], buf, sem).wait()
            pltpu.sync_copy(buf, out_h.at[pl.ds(base + i * _GCH, _GCH)])
            return carry

        lax.fori_loop(0, _GNCH, step, 0)

    return gk(table, idx_all)


_NWK = 32
_OWN = 10240
_EPAD = _NWK * _OWN
_A = 320
_NP2 = _OWN // _A
_SENT = 1 << 20
_C1CH = 6272
_L0 = 100352
_L1 = 125440
_CAP0 = 100480
_CAP1 = 125568
_SCH = 48
_LCH = 2048


def _sc_route(idx0p, idx1p):
    mesh = plsc.VectorSubcoreMesh(core_axis_name="c", subcore_axis_name="s")

    @functools.partial(
        pl.kernel, mesh=mesh,
        compiler_params=pltpu.CompilerParams(needs_layout_passes=False),
        out_type=(jax.ShapeDtypeStruct((_NWK * _CAP0,), jnp.int32),
                  jax.ShapeDtypeStruct((_NWK * _CAP1,), jnp.int32),
                  jax.ShapeDtypeStruct((_NWK, 16), jnp.int32)),
        scratch_types=[
            pltpu.VMEM((_C1CH,), jnp.int32),
            pltpu.VMEM((256,), jnp.int32),
            pltpu.VMEM((128,), jnp.int32),
            pltpu.VMEM((16,), jnp.int32),
            pltpu.SemaphoreType.DMA,
        ],
    )
    def rk(idx0_h, idx1_h, l0_h, l1_h, cnt_h, ichunk, ebuf, posb, cbuf, sem):
        wid = lax.axis_index("s") * 2 + lax.axis_index("c")
        wbase = wid * _OWN
        lanes = lax.iota(jnp.int32, 16)

        def scan_array(idx_h, nchunks, l_h, lcap):
            def chunk_body(ci, carry):
                fill, total = carry
                pltpu.sync_copy(idx_h.at[pl.ds(ci * _C1CH, _C1CH)], ichunk)

                def vbody(v, carry2):
                    fill2, total2 = carry2
                    iv = ichunk[pl.ds(v * 16, 16)]
                    m = (iv >= wbase) & (iv < wbase + _OWN)
                    mi = m.astype(jnp.int32)
                    cs = plsc.cumsum(mi)
                    pos = fill2 + cs - mi
                    rid = ci * _C1CH + v * 16 + lanes
                    packed = rid * 16384 + (iv - wbase)
                    plsc.store_scatter(ebuf, [pos], packed, mask=m)
                    fill2 = fill2 + cs[15]

                    def flush(args):
                        f, t = args
                        for k in range(8):
                            posb[pl.ds(k * 16, 16)] = (
                                wid * lcap + t + k * 16 + lanes)
                        pltpu.async_copy(ebuf.at[pl.ds(0, 128)],
                                         l_h.at[posb], sem).wait()
                        tail = ebuf[pl.ds(128, 16)]
                        ebuf[pl.ds(0, 16)] = tail
                        return f - 128, t + 128

                    fill2, total2 = lax.cond(
                        fill2 >= 128, flush, lambda a: a, (fill2, total2))
                    return fill2, total2

                return lax.fori_loop(0, _C1CH // 16, vbody, (fill, total))

            fill, total = lax.fori_loop(0, nchunks, chunk_body, (0, 0))
            for k in range(8):
                posb[pl.ds(k * 16, 16)] = wid * lcap + total + k * 16 + lanes
            pltpu.async_copy(ebuf.at[pl.ds(0, 128)], l_h.at[posb], sem).wait()
            return total + fill

        n0 = scan_array(idx0_h, _L0 // _C1CH, l0_h, _CAP0)
        n1 = scan_array(idx1_h, _L1 // _C1CH, l1_h, _CAP1)
        cbuf[pl.ds(0, 16)] = n0 * (lanes < 1) + n1 * ((lanes >= 1) & (lanes < 2))
        pltpu.sync_copy(cbuf, cnt_h.at[wid])

    return rk(idx0p, idx1p)


def _sc_accum(list0, list1, counts, lm0, lm1):
    mesh = plsc.VectorSubcoreMesh(core_axis_name="c", subcore_axis_name="s")

    @functools.partial(
        pl.kernel, mesh=mesh,
        compiler_params=pltpu.CompilerParams(needs_layout_passes=False),
        out_type=jax.ShapeDtypeStruct((_EPAD, 2 * _REP), jnp.float32),
        scratch_types=[
            pltpu.VMEM((_A + 16, 2 * _REP), jnp.float32),
            pltpu.VMEM((_SCH, 2 * _REP), jnp.float32),
            pltpu.VMEM((_SCH, 2 * _REP), jnp.float32),
            pltpu.VMEM((_LCH,), jnp.int32),
            pltpu.VMEM((_LCH,), jnp.int32),
            pltpu.VMEM((_LCH + _SCH,), jnp.int32),
            pltpu.VMEM((_LCH + _SCH,), jnp.int32),
            pltpu.VMEM((_SCH,), jnp.int32),
            pltpu.VMEM((_SCH,), jnp.int32),
            pltpu.VMEM((_SCH,), jnp.int32),
            pltpu.VMEM((_SCH,), jnp.int32),
            pltpu.VMEM((16,), jnp.int32),
            pltpu.SemaphoreType.DMA,
            pltpu.SemaphoreType.DMA,
            pltpu.SemaphoreType.DMA,
            pltpu.SemaphoreType.DMA,
        ],
    )
    def ak(l0_h, l1_h, cnt_h, lm0_h, lm1_h, out_h,
           acc, stageA, stageB, lbufA, lbufB, srcl, dstl,
           srcbA, dstbA, srcbB, dstbB, cbuf, semA, semB, semLA, semLB):
        wid = lax.axis_index("s") * 2 + lax.axis_index("c")
        lanes = lax.iota(jnp.int32, 16)
        ones16 = jnp.ones((16,), jnp.bool_)
        pltpu.sync_copy(cnt_h.at[wid], cbuf)
        cv = cbuf[pl.ds(0, 16)]
        n0 = cv[0]
        n1 = cv[1]

        def process(l_h, lcap, n, lm_h, lo):
            nlc = (n + _LCH - 1) // _LCH
            lbase = wid * lcap

            @pl.when(nlc > 0)
            def _():
                pltpu.async_copy(l_h.at[pl.ds(lbase, _LCH)], lbufA, semLA)

            def lchunk_work(ci, lbuf, semL, lbufN, semLN):
                @pl.when(ci + 1 < nlc)
                def _():
                    pltpu.async_copy(
                        l_h.at[pl.ds(lbase + (ci + 1) * _LCH, _LCH)],
                        lbufN, semLN)
                pltpu.make_async_copy(l_h.at[pl.ds(lbase, _LCH)], lbuf,
                                      semL).wait()

                def vbody(v, nm):
                    ev = lbuf[pl.ds(v * 16, 16)]
                    rid = lax.shift_right_logical(ev, 14)
                    loc = ev & 16383
                    valid = (ci * _LCH + v * 16 + lanes) < n
                    m = valid & (loc >= lo) & (loc < lo + _A)
                    mi = m.astype(jnp.int32)
                    cs = plsc.cumsum(mi)
                    pos = nm + cs - mi
                    plsc.store_scatter(srcl, [pos], rid, mask=m)
                    plsc.store_scatter(dstl, [pos], loc - lo, mask=m)
                    return nm + cs[15]

                nm = lax.fori_loop(0, _LCH // 16, vbody, 0)
                for k in range(_SCH // 16):
                    plsc.store_scatter(srcl, [nm + k * 16 + lanes],
                                       jnp.zeros((16,), jnp.int32), mask=ones16)
                    plsc.store_scatter(dstl, [nm + k * 16 + lanes],
                                       jnp.full((16,), _A, jnp.int32),
                                       mask=ones16)
                ngc = (nm + _SCH - 1) // _SCH

                def fire(i, srcb, dstb, stage, sem):
                    for k in range(_SCH // 16):
                        srcb[pl.ds(k * 16, 16)] = srcl[pl.ds(i * _SCH + k * 16, 16)]
                        dstb[pl.ds(k * 16, 16)] = dstl[pl.ds(i * _SCH + k * 16, 16)]
                    pltpu.async_copy(lm_h.at[srcb], stage, sem)

                def drain_process(dstb, stage, sem):
                    pltpu.make_async_copy(lm_h.at[pl.ds(0, _SCH)], stage,
                                          sem).wait()

                    def grp(g, c3):
                        dv = dstb[pl.ds(g * 16, 16)]
                        for lane in range(16):
                            d = dv[lane]
                            for j in range(16):
                                plsc.addupdate(
                                    acc.at[d, pl.ds(j * 16, 16)],
                                    stage[g * 16 + lane, pl.ds(j * 16, 16)])
                        return c3

                    lax.fori_loop(0, _SCH // 16, grp, 0)

                @pl.when(ngc > 0)
                def _():
                    fire(0, srcbA, dstbA, stageA, semA)

                def gs(i, c2):
                    def even(_):
                        @pl.when(i + 1 < ngc)
                        def _():
                            fire(i + 1, srcbB, dstbB, stageB, semB)
                        drain_process(dstbA, stageA, semA)
                        return 0

                    def odd(_):
                        @pl.when(i + 1 < ngc)
                        def _():
                            fire(i + 1, srcbA, dstbA, stageA, semA)
                        drain_process(dstbB, stageB, semB)
                        return 0

                    lax.cond((i & 1) == 0, even, odd, 0)
                    return c2

                lax.fori_loop(0, ngc, gs, 0)

            def lchunk(ci, carry):
                def evenc(_):
                    lchunk_work(ci, lbufA, semLA, lbufB, semLB)
                    return 0

                def oddc(_):
                    lchunk_work(ci, lbufB, semLB, lbufA, semLA)
                    return 0

                lax.cond((ci & 1) == 0, evenc, oddc, 0)
                return carry

            lax.fori_loop(0, nlc, lchunk, 0)

        def pass_body(p, carry):
            zv = jnp.zeros((16,), jnp.float32)

            def zrow(r, c):
                for rr in range(4):
                    for j in range(16):
                        acc[r * 4 + rr, pl.ds(j * 16, 16)] = zv
                return c

            lax.fori_loop(0, _A // 4, zrow, 0)
            lo = p * _A
            process(l0_h, _CAP0, n0, lm0_h, lo)
            process(l1_h, _CAP1, n1, lm1_h, lo)
            pltpu.sync_copy(acc.at[pl.ds(0, _A)],
                            out_h.at[pl.ds(wid * _OWN + p * _A, _A)])
            return carry

        lax.fori_loop(0, _NP2, pass_body, 0)

    return ak(list0, list1, counts, lm0, lm1)


def kernel(edge_rep, cycles_rep_0, cycles_rep_1, e2c_idx_0, e2c_idx_1,
           cmlp_W1, cmlp_b1, cmlp_W2, cmlp_b2,
           emlp_W1, emlp_b1, emlp_W2, emlp_b2,
           a1n_0, a1s_0, a1b_0, a1n_1, a1s_1, a1b_1,
           a2n_0, a2s_0, a2b_0, a2n_1, a2s_1, a2b_1):
    idx_all = jnp.concatenate(
        [e2c_idx_0, e2c_idx_1,
         jnp.zeros((_GN - _R0 - _R1,), jnp.int32)])
    g_all = _sc_gather(edge_rep, idx_all)
    g0 = g_all[0:_R0]
    g1 = g_all[_R0:_R0 + _R1]
    lm0 = _family_call(5, _R0, cycles_rep_0, g0, cmlp_W1, cmlp_b1, cmlp_W2,
                       cmlp_b2, a1n_0, a1s_0, a1b_0, a2n_0, a2s_0, a2b_0)
    lm1 = _family_call(6, _R1, cycles_rep_1, g1, cmlp_W1, cmlp_b1, cmlp_W2,
                       cmlp_b2, a1n_1, a1s_1, a1b_1, a2n_1, a2s_1, a2b_1)
    idx0p = jnp.concatenate(
        [e2c_idx_0, jnp.full((_L0 - _R0,), _SENT, jnp.int32)])
    idx1p = jnp.concatenate(
        [e2c_idx_1, jnp.full((_L1 - _R1,), _SENT, jnp.int32)])
    list0, list1, counts = _sc_route(idx0p, idx1p)
    c2e = _sc_accum(list0, list1, counts, lm0, lm1)
    edge_out = _edge_call(edge_rep, c2e, emlp_W1, emlp_b1, emlp_W2, emlp_b2)
    return edge_out, lm0[:, 0:_REP], lm1[:, 0:_REP]


# speedup vs baseline: 1.0075x; 1.0075x over previous
import functools

import jax
import jax.numpy as jnp
from jax import lax
from jax.experimental import pallas as pl
from jax.experimental.pallas import tpu as pltpu
from jax.experimental.pallas import tpu_sc as plsc

_REP = 128
_E = 320000
_NSEG = 20000
_R0 = 100000
_R1 = 120000

_BSEG = 80


def _mmf():
    return functools.partial(
        lax.dot_general,
        dimension_numbers=(((1,), (0,)), ((), ())),
        preferred_element_type=jnp.float32,
    )


def _family_body(size):
    brows = _BSEG * size

    def body(c_ref, g_ref, w1_ref, b1_ref, w2_ref, b2_ref,
             a1n_ref, a1s_ref, a1b_ref, a2n_ref, a2s_ref, a2b_ref, lm_ref):
        mm = _mmf()
        c = c_ref[...]
        g = g_ref[...]
        ri = lax.broadcasted_iota(jnp.int32, (brows, brows), 0) // size
        ci = lax.broadcasted_iota(jnp.int32, (brows, brows), 1) // size
        p = (ri == ci).astype(jnp.float32)
        w1 = w1_ref[...]
        hp = (mm(c, w1[0:128]) + mm(mm(p, c), w1[128:256])
              + mm(g, w1[256:384]) + mm(mm(p, g), w1[384:512]) + b1_ref[...])
        h = mm(jnp.maximum(hp, 0.0), w2_ref[...]) + b2_ref[...]
        a = jnp.maximum(
            mm(h, a1n_ref[...]) + mm(mm(p, h), a1s_ref[...]) + a1b_ref[...], 0.0)
        o = jnp.maximum(
            mm(a, a2n_ref[...]) + mm(mm(p, a), a2s_ref[...]) + a2b_ref[...], 0.0)
        lm_ref[:, 0:128] = o
        lm_ref[:, 128:256] = mm(p, o)

    return body


def _family_call(size, rows, c, g, w1, b1, w2, b2, a1n, a1s, a1b, a2n, a2s, a2b):
    brows = _BSEG * size
    row_spec = pl.BlockSpec((brows, _REP), lambda i: (i, 0))

    def wspec(shape):
        return pl.BlockSpec(shape, lambda i: (0,) * len(shape))

    return pl.pallas_call(
        _family_body(size),
        grid=(_NSEG // _BSEG,),
        in_specs=[row_spec, row_spec,
                  wspec((4 * _REP, 2 * _REP)), wspec((1, 2 * _REP)),
                  wspec((2 * _REP, _REP)), wspec((1, _REP)),
                  wspec((_REP, 2 * _REP)), wspec((_REP, 2 * _REP)),
                  wspec((1, 2 * _REP)),
                  wspec((2 * _REP, _REP)), wspec((2 * _REP, _REP)),
                  wspec((1, _REP))],
        out_specs=pl.BlockSpec((brows, 2 * _REP), lambda i: (i, 0)),
        out_shape=jax.ShapeDtypeStruct((rows, 2 * _REP), jnp.float32),
    )(c, g, w1, b1.reshape(1, -1), w2, b2.reshape(1, -1),
      a1n, a1s, a1b.reshape(1, -1), a2n, a2s, a2b.reshape(1, -1))


_BE = 512


def _edge_body(e_ref, m_ref, w1_ref, b1_ref, w2_ref, b2_ref, o_ref):
    mm = _mmf()
    w1 = w1_ref[...]
    hp = mm(e_ref[...], w1[0:128]) + mm(m_ref[...], w1[128:384]) + b1_ref[...]
    o_ref[...] = mm(jnp.maximum(hp, 0.0), w2_ref[...]) + b2_ref[...]


def _edge_call(edge_rep, c2e, w1, b1, w2, b2):
    return pl.pallas_call(
        _edge_body,
        grid=(_E // _BE,),
        in_specs=[pl.BlockSpec((_BE, _REP), lambda i: (i, 0)),
                  pl.BlockSpec((_BE, 2 * _REP), lambda i: (i, 0)),
                  pl.BlockSpec((3 * _REP, 2 * _REP), lambda i: (0, 0)),
                  pl.BlockSpec((1, 2 * _REP), lambda i: (0, 0)),
                  pl.BlockSpec((2 * _REP, _REP), lambda i: (0, 0)),
                  pl.BlockSpec((1, _REP), lambda i: (0, 0))],
        out_specs=pl.BlockSpec((_BE, _REP), lambda i: (i, 0)),
        out_shape=jax.ShapeDtypeStruct((_E, _REP), jnp.float32),
    )(edge_rep, c2e, w1, b1.reshape(1, -1), w2, b2.reshape(1, -1))


_GN = 225280
_GPT = _GN // 32
_GCH = 128
_GNCH = _GPT // _GCH


def _sc_gather(table, idx_all):
    mesh = plsc.VectorSubcoreMesh(core_axis_name="c", subcore_axis_name="s")

    @functools.partial(
        pl.kernel, mesh=mesh,
        compiler_params=pltpu.CompilerParams(needs_layout_passes=False),
        out_type=jax.ShapeDtypeStruct((_GN, _REP), jnp.float32),
        scratch_types=[pltpu.VMEM((_GPT,), jnp.int32),
                       pltpu.VMEM((_GCH,), jnp.int32),
                       pltpu.VMEM((_GCH, _REP), jnp.float32),
                       pltpu.SemaphoreType.DMA],
    )
    def gk(table_h, idx_h, out_h, idx_v, idxb, buf, sem):
        wid = lax.axis_index("s") * 2 + lax.axis_index("c")
        base = wid * _GPT
        pltpu.sync_copy(idx_h.at[pl.ds(base, _GPT)], idx_v)

        def step(i, carry):
            for k in range(_GCH // 16):
                idxb[pl.ds(k * 16, 16)] = idx_v[pl.ds(i * _GCH + k * 16, 16)]
            pltpu.async_copy(table_h.at[idxb], buf, sem).wait()
            pltpu.sync_copy(buf, out_h.at[pl.ds(base + i * _GCH, _GCH)])
            return carry

        lax.fori_loop(0, _GNCH, step, 0)

    return gk(table, idx_all)


_NWK = 32
_OWN = 10240
_EPAD = _NWK * _OWN
_A = 320
_NP2 = _OWN // _A
_SENT = 1 << 20
_C1CH = 6272
_L0 = 100352
_L1 = 125440
_CAP0 = 100480
_CAP1 = 125568
_SCH = 48
_LCH = 2048


def _sc_route(idx0p, idx1p):
    mesh = plsc.VectorSubcoreMesh(core_axis_name="c", subcore_axis_name="s")

    @functools.partial(
        pl.kernel, mesh=mesh,
        compiler_params=pltpu.CompilerParams(needs_layout_passes=False),
        out_type=(jax.ShapeDtypeStruct((_NWK * _CAP0,), jnp.int32),
                  jax.ShapeDtypeStruct((_NWK * _CAP1,), jnp.int32),
                  jax.ShapeDtypeStruct((_NWK, 16), jnp.int32)),
        scratch_types=[
            pltpu.VMEM((_C1CH,), jnp.int32),
            pltpu.VMEM((256,), jnp.int32),
            pltpu.VMEM((128,), jnp.int32),
            pltpu.VMEM((16,), jnp.int32),
            pltpu.SemaphoreType.DMA,
        ],
    )
    def rk(idx0_h, idx1_h, l0_h, l1_h, cnt_h, ichunk, ebuf, posb, cbuf, sem):
        wid = lax.axis_index("s") * 2 + lax.axis_index("c")
        wbase = wid * _OWN
        lanes = lax.iota(jnp.int32, 16)

        def scan_array(idx_h, nchunks, l_h, lcap):
            def chunk_body(ci, carry):
                fill, total = carry
                pltpu.sync_copy(idx_h.at[pl.ds(ci * _C1CH, _C1CH)], ichunk)

                def vbody(v, carry2):
                    fill2, total2 = carry2
                    iv = ichunk[pl.ds(v * 16, 16)]
                    m = (iv >= wbase) & (iv < wbase + _OWN)
                    mi = m.astype(jnp.int32)
                    cs = plsc.cumsum(mi)
                    pos = fill2 + cs - mi
                    rid = ci * _C1CH + v * 16 + lanes
                    packed = rid * 16384 + (iv - wbase)
                    plsc.store_scatter(ebuf, [pos], packed, mask=m)
                    fill2 = fill2 + cs[15]

                    def flush(args):
                        f, t = args
                        for k in range(8):
                            posb[pl.ds(k * 16, 16)] = (
                                wid * lcap + t + k * 16 + lanes)
                        pltpu.async_copy(ebuf.at[pl.ds(0, 128)],
                                         l_h.at[posb], sem).wait()
                        tail = ebuf[pl.ds(128, 16)]
                        ebuf[pl.ds(0, 16)] = tail
                        return f - 128, t + 128

                    fill2, total2 = lax.cond(
                        fill2 >= 128, flush, lambda a: a, (fill2, total2))
                    return fill2, total2

                return lax.fori_loop(0, _C1CH // 16, vbody, (fill, total))

            fill, total = lax.fori_loop(0, nchunks, chunk_body, (0, 0))
            for k in range(8):
                posb[pl.ds(k * 16, 16)] = wid * lcap + total + k * 16 + lanes
            pltpu.async_copy(ebuf.at[pl.ds(0, 128)], l_h.at[posb], sem).wait()
            return total + fill

        n0 = scan_array(idx0_h, _L0 // _C1CH, l0_h, _CAP0)
        n1 = scan_array(idx1_h, _L1 // _C1CH, l1_h, _CAP1)
        cbuf[pl.ds(0, 16)] = n0 * (lanes < 1) + n1 * ((lanes >= 1) & (lanes < 2))
        pltpu.sync_copy(cbuf, cnt_h.at[wid])

    return rk(idx0p, idx1p)


def _sc_accum(list0, list1, counts, lm0, lm1):
    mesh = plsc.VectorSubcoreMesh(core_axis_name="c", subcore_axis_name="s")

    @functools.partial(
        pl.kernel, mesh=mesh,
        compiler_params=pltpu.CompilerParams(needs_layout_passes=False),
        out_type=jax.ShapeDtypeStruct((_EPAD, 2 * _REP), jnp.float32),
        scratch_types=[
            pltpu.VMEM((_A + 16, 2 * _REP), jnp.float32),
            pltpu.VMEM((_SCH, 2 * _REP), jnp.float32),
            pltpu.VMEM((_SCH, 2 * _REP), jnp.float32),
            pltpu.VMEM((_LCH,), jnp.int32),
            pltpu.VMEM((_LCH,), jnp.int32),
            pltpu.VMEM((_LCH + _SCH,), jnp.int32),
            pltpu.VMEM((_LCH + _SCH,), jnp.int32),
            pltpu.VMEM((_SCH,), jnp.int32),
            pltpu.VMEM((_SCH,), jnp.int32),
            pltpu.VMEM((_SCH,), jnp.int32),
            pltpu.VMEM((_SCH,), jnp.int32),
            pltpu.VMEM((16,), jnp.int32),
            pltpu.SemaphoreType.DMA,
            pltpu.SemaphoreType.DMA,
            pltpu.SemaphoreType.DMA,
            pltpu.SemaphoreType.DMA,
        ],
    )
    def ak(l0_h, l1_h, cnt_h, lm0_h, lm1_h, out_h,
           acc, stageA, stageB, lbufA, lbufB, srcl, dstl,
           srcbA, dstbA, srcbB, dstbB, cbuf, semA, semB, semLA, semLB):
        wid = lax.axis_index("s") * 2 + lax.axis_index("c")
        lanes = lax.iota(jnp.int32, 16)
        ones16 = jnp.ones((16,), jnp.bool_)
        pltpu.sync_copy(cnt_h.at[wid], cbuf)
        cv = cbuf[pl.ds(0, 16)]
        n0 = cv[0]
        n1 = cv[1]

        def process(l_h, lcap, n, lm_h, lo):
            nlc = (n + _LCH - 1) // _LCH
            lbase = wid * lcap

            @pl.when(nlc > 0)
            def _():
                pltpu.async_copy(l_h.at[pl.ds(lbase, _LCH)], lbufA, semLA)

            def lchunk_work(ci, lbuf, semL, lbufN, semLN):
                @pl.when(ci + 1 < nlc)
                def _():
                    pltpu.async_copy(
                        l_h.at[pl.ds(lbase + (ci + 1) * _LCH, _LCH)],
                        lbufN, semLN)
                pltpu.make_async_copy(l_h.at[pl.ds(lbase, _LCH)], lbuf,
                                      semL).wait()

                def vbody(v, nm_c):
                    ev = lbuf[pl.ds(v * 16, 16)]
                    rid = lax.shift_right_logical(ev, 14)
                    loc = ev & 16383
                    valid = (ci * _LCH + v * 16 + lanes) < n
                    m = valid & (loc >= lo) & (loc < lo + _A)
                    mi = m.astype(jnp.int32)
                    cs = plsc.cumsum(mi)
                    pos = nm_c + cs - mi
                    plsc.store_scatter(srcl, [pos], rid, mask=m)
                    plsc.store_scatter(dstl, [pos], loc - lo, mask=m)
                    return nm_c + cs[15]

                nm = lax.fori_loop(0, _LCH // 16, vbody, 0)
                for k in range(_SCH // 16):
                    plsc.store_scatter(srcl, [nm + k * 16 + lanes],
                                       jnp.zeros((16,), jnp.int32), mask=ones16)
                    plsc.store_scatter(dstl, [nm + k * 16 + lanes],
                                       jnp.full((16,), _A, jnp.int32),
                                       mask=ones16)
                ngc = (nm + _SCH - 1) // _SCH

                def fire(i, srcb, dstb, stage, sem):
                    for k in range(_SCH // 16):
                        srcb[pl.ds(k * 16, 16)] = srcl[pl.ds(i * _SCH + k * 16, 16)]
                        dstb[pl.ds(k * 16, 16)] = dstl[pl.ds(i * _SCH + k * 16, 16)]
                    pltpu.async_copy(lm_h.at[srcb], stage, sem)

                def drain_process(dstb, stage, sem):
                    pltpu.make_async_copy(lm_h.at[pl.ds(0, _SCH)], stage,
                                          sem).wait()

                    def grp(g, c3):
                        dv = dstb[pl.ds(g * 16, 16)]
                        for lane in range(16):
                            d = dv[lane]
                            for j in range(16):
                                plsc.addupdate(
                                    acc.at[d, pl.ds(j * 16, 16)],
                                    stage[g * 16 + lane, pl.ds(j * 16, 16)])
                        return c3

                    lax.fori_loop(0, _SCH // 16, grp, 0)

                @pl.when(ngc > 0)
                def _():
                    fire(0, srcbA, dstbA, stageA, semA)

                def gs(i, c2):
                    def even(_):
                        @pl.when(i + 1 < ngc)
                        def _():
                            fire(i + 1, srcbB, dstbB, stageB, semB)
                        drain_process(dstbA, stageA, semA)
                        return 0

                    def odd(_):
                        @pl.when(i + 1 < ngc)
                        def _():
                            fire(i + 1, srcbA, dstbA, stageA, semA)
                        drain_process(dstbB, stageB, semB)
                        return 0

                    lax.cond((i & 1) == 0, even, odd, 0)
                    return c2

                lax.fori_loop(0, ngc, gs, 0)

            def lchunk(ci, carry):
                def evenc(_):
                    lchunk_work(ci, lbufA, semLA, lbufB, semLB)
                    return 0

                def oddc(_):
                    lchunk_work(ci, lbufB, semLB, lbufA, semLA)
                    return 0

                lax.cond((ci & 1) == 0, evenc, oddc, 0)
                return carry

            lax.fori_loop(0, nlc, lchunk, 0)

        def pass_body(p, carry):
            zv = jnp.zeros((16,), jnp.float32)

            def zrow(r, c):
                for rr in range(4):
                    for j in range(16):
                        acc[r * 4 + rr, pl.ds(j * 16, 16)] = zv
                return c

            lax.fori_loop(0, _A // 4, zrow, 0)
            lo = p * _A
            process(l0_h, _CAP0, n0, lm0_h, lo)
            process(l1_h, _CAP1, n1, lm1_h, lo)
            pltpu.sync_copy(acc.at[pl.ds(0, _A)],
                            out_h.at[pl.ds(wid * _OWN + p * _A, _A)])
            return carry

        lax.fori_loop(0, _NP2, pass_body, 0)

    return ak(list0, list1, counts, lm0, lm1)


def kernel(edge_rep, cycles_rep_0, cycles_rep_1, e2c_idx_0, e2c_idx_1,
           cmlp_W1, cmlp_b1, cmlp_W2, cmlp_b2,
           emlp_W1, emlp_b1, emlp_W2, emlp_b2,
           a1n_0, a1s_0, a1b_0, a1n_1, a1s_1, a1b_1,
           a2n_0, a2s_0, a2b_0, a2n_1, a2s_1, a2b_1):
    idx_all = jnp.concatenate(
        [e2c_idx_0, e2c_idx_1,
         jnp.zeros((_GN - _R0 - _R1,), jnp.int32)])
    g_all = _sc_gather(edge_rep, idx_all)
    g0 = g_all[0:_R0]
    g1 = g_all[_R0:_R0 + _R1]
    lm0 = _family_call(5, _R0, cycles_rep_0, g0, cmlp_W1, cmlp_b1, cmlp_W2,
                       cmlp_b2, a1n_0, a1s_0, a1b_0, a2n_0, a2s_0, a2b_0)
    lm1 = _family_call(6, _R1, cycles_rep_1, g1, cmlp_W1, cmlp_b1, cmlp_W2,
                       cmlp_b2, a1n_1, a1s_1, a1b_1, a2n_1, a2s_1, a2b_1)
    idx0p = jnp.concatenate(
        [e2c_idx_0, jnp.full((_L0 - _R0,), _SENT, jnp.int32)])
    idx1p = jnp.concatenate(
        [e2c_idx_1, jnp.full((_L1 - _R1,), _SENT, jnp.int32)])
    list0, list1, counts = _sc_route(idx0p, idx1p)
    c2e = _sc_accum(list0, list1, counts, lm0, lm1)
    edge_out = _edge_call(edge_rep, c2e, emlp_W1, emlp_b1, emlp_W2, emlp_b2)
    return edge_out, lm0[:, 0:_REP], lm1[:, 0:_REP]
